# Initial kernel scaffold; baseline (speedup 1.0000x reference)
import jax
import jax.numpy as jnp
from jax.experimental import pallas as pl


def kernel(x, tables, cvr_w1, cvr_b1, cvr_w2, cvr_b2, cvr_w3, cvr_b3, ctr_w1, ctr_b1, ctr_w2, ctr_b2, ctr_w3, ctr_b3):
    raise NotImplementedError("write your pallas kernel here")


# speedup vs baseline: 1.0001x; 1.0001x over previous
import functools

import jax
import jax.numpy as jnp
from jax import lax
from jax.experimental import pallas as pl
from jax.experimental.pallas import tpu as pltpu
from jax.experimental.pallas import tpu_sc as plsc

B = 16384
F = 26
F_USER = 13
V = 100000
D = 32
TOWER_IN = 2 * D
H1, H2 = 256, 128

NC = 2
NS = 16
NW = NC * NS
ROWS_W = B // NW
CB = 64
IDX_CB = CB * F
NSUB = IDX_CB // 128
CHUNKS = ROWS_W // CB


def _sc_pool(xflat2d, table_flat):
  mesh = plsc.VectorSubcoreMesh(core_axis_name="c", subcore_axis_name="s")

  idx_rows_w = ROWS_W * F // 128

  @functools.partial(
      pl.kernel,
      mesh=mesh,
      compiler_params=pltpu.CompilerParams(use_tc_tiling_on_sc=False),
      out_type=jax.ShapeDtypeStruct((B, TOWER_IN), jnp.float32),
      scratch_types=[
          pltpu.VMEM((idx_rows_w, 128), jnp.int32),
          pltpu.VMEM((IDX_CB, D), jnp.float32),
          pltpu.VMEM((CB, TOWER_IN), jnp.float32),
          pltpu.SemaphoreType.DMA,
      ],
  )
  def k(x_hbm, t_hbm, out_hbm, idx_v, rows_v, pool_v, sem):
    wid = lax.axis_index("s") * NC + lax.axis_index("c")
    base = pl.multiple_of(wid * ROWS_W, ROWS_W)
    irow = pl.multiple_of(wid * idx_rows_w, 8)
    pltpu.sync_copy(x_hbm.at[pl.ds(irow, idx_rows_w)], idx_v)

    for ci in range(CHUNKS):
      start = base + ci * CB
      copies = [
          pltpu.async_copy(t_hbm.at[idx_v.at[ci * NSUB + j]],
                           rows_v.at[pl.ds(j * 128, 128)], sem)
          for j in range(NSUB)
      ]
      for cp in copies:
        cp.wait()

      def row(r, c2):
        rb = r * F
        for half in (0, 1):
          dsl = pl.ds(half * 16, 16)
          u = rows_v[rb, dsl]
          for f in range(1, F_USER):
            u = u + rows_v[rb + f, dsl]
          it = rows_v[rb + F_USER, dsl]
          for f in range(F_USER + 1, F):
            it = it + rows_v[rb + f, dsl]
          pool_v[r, pl.ds(half * 16, 16)] = u
          pool_v[r, pl.ds(D + half * 16, 16)] = it
        return c2

      lax.fori_loop(0, CB, row, 0)
      pltpu.sync_copy(pool_v, out_hbm.at[pl.ds(start, CB)])

  return k(xflat2d, table_flat)


BS = 1024


def _mlp_body(x_ref, cw1, cb1, cw2, cb2, cw3, cb3,
              tw1, tb1, tw2, tb2, tw3, tb3, out_ref):
  h = x_ref[...]

  def tower(w1, b1, w2, b2, w3, b3):
    h1 = jnp.maximum(
        jnp.dot(h, w1[...], preferred_element_type=jnp.float32) + b1[...], 0.0)
    h2 = jnp.maximum(
        jnp.dot(h1, w2[...], preferred_element_type=jnp.float32) + b2[...], 0.0)
    return jnp.dot(h2, w3[...], preferred_element_type=jnp.float32) + b3[...]

  cvr = jax.nn.sigmoid(tower(cw1, cb1, cw2, cb2, cw3, cb3))
  ctr = jax.nn.sigmoid(tower(tw1, tb1, tw2, tb2, tw3, tb3))
  out_ref[...] = jnp.concatenate([cvr, ctr, cvr * cvr], axis=1)


def _tc_mlp(pooled, *weights):
  def full(shape):
    return pl.BlockSpec(shape, lambda i: (0, 0))

  wspecs = [
      full((TOWER_IN, H1)), full((1, H1)),
      full((H1, H2)), full((1, H2)),
      full((H2, 1)), full((1, 1)),
  ] * 2
  return pl.pallas_call(
      _mlp_body,
      grid=(B // BS,),
      in_specs=[pl.BlockSpec((BS, TOWER_IN), lambda i: (i, 0))] + wspecs,
      out_specs=pl.BlockSpec((BS, 3), lambda i: (i, 0)),
      out_shape=jax.ShapeDtypeStruct((B, 3), jnp.float32),
  )(pooled, *weights)


def kernel(x, tables, cvr_w1, cvr_b1, cvr_w2, cvr_b2, cvr_w3, cvr_b3,
           ctr_w1, ctr_b1, ctr_w2, ctr_b2, ctr_w3, ctr_b3):
  xi = x.astype(jnp.int32)
  xflat = (xi + jnp.arange(F, dtype=jnp.int32)[None, :] * V)
  xflat2d = xflat.reshape(B * F // 128, 128)
  table_flat = tables.reshape(F * V, D)
  pooled = _sc_pool(xflat2d, table_flat)
  return _tc_mlp(
      pooled,
      cvr_w1, cvr_b1.reshape(1, H1), cvr_w2, cvr_b2.reshape(1, H2),
      cvr_w3, cvr_b3.reshape(1, 1),
      ctr_w1, ctr_b1.reshape(1, H1), ctr_w2, ctr_b2.reshape(1, H2),
      ctr_w3, ctr_b3.reshape(1, 1))
